# baseline (speedup 1.0000x reference)
import functools

import jax
import jax.numpy as jnp
from jax import lax
from jax.experimental import pallas as pl
from jax.experimental.pallas import tpu as pltpu
from jax.experimental.pallas import tpu_sc as plsc

N = 10000
E = 320000
IN = 128
ED = 16
H = 8
C = 16
HID = H * C
OUT = 40

NC = 2
NS = 16
NW = NC * NS
EPAD = 327680
EH = EPAD // 2
EPWH = EH // NW
CHUNK = 128
NCHUNKH = EPWH // CHUNK
NPADN = 10112
NPT = NPADN // NS
NZS = (128, 128, 128, 128, 120)

BE = 2560
EGRIDH = EH // BE

_f32 = jnp.float32


def _sc_gather_body(xl_hbm, xr_hbm, src2_hbm, dst2_hbm, xls_out, xrd_out,
                    idxs_v, idxd_v, a0, a1, b0, b1, sa0, sa1, sb0, sb1):
    cid = lax.axis_index("c")
    sid = lax.axis_index("s")
    wid = cid * NS + sid
    tb = wid * EPWH

    pltpu.sync_copy(src2_hbm.at[pl.ds(wid * NCHUNKH, NCHUNKH)], idxs_v)
    pltpu.sync_copy(dst2_hbm.at[pl.ds(wid * NCHUNKH, NCHUNKH)], idxd_v)

    def start(j, buf, sem, table, idx):
        pltpu.async_copy(table.at[idx.at[j]], buf, sem)

    def drain(buf, sem, table, idx):
        pltpu.make_async_copy(table.at[idx.at[0]], buf, sem).wait()

    start(0, a0, sa0, xl_hbm, idxs_v)
    start(0, b0, sb0, xr_hbm, idxd_v)
    start(1, a1, sa1, xl_hbm, idxs_v)
    start(1, b1, sb1, xr_hbm, idxd_v)

    def body(t, carry):
        j0 = 2 * t
        j1 = j0 + 1
        drain(a0, sa0, xl_hbm, idxs_v)
        pltpu.sync_copy(a0, xls_out.at[pl.ds(tb + j0 * CHUNK, CHUNK)])
        drain(b0, sb0, xr_hbm, idxd_v)
        pltpu.sync_copy(b0, xrd_out.at[pl.ds(tb + j0 * CHUNK, CHUNK)])

        @pl.when(j0 + 2 < NCHUNKH)
        def _():
            start(j0 + 2, a0, sa0, xl_hbm, idxs_v)
            start(j0 + 2, b0, sb0, xr_hbm, idxd_v)

        drain(a1, sa1, xl_hbm, idxs_v)
        pltpu.sync_copy(a1, xls_out.at[pl.ds(tb + j1 * CHUNK, CHUNK)])
        drain(b1, sb1, xr_hbm, idxd_v)
        pltpu.sync_copy(b1, xrd_out.at[pl.ds(tb + j1 * CHUNK, CHUNK)])

        @pl.when(j1 + 2 < NCHUNKH)
        def _():
            start(j1 + 2, a1, sa1, xl_hbm, idxs_v)
            start(j1 + 2, b1, sb1, xr_hbm, idxd_v)

        return carry

    lax.fori_loop(0, NCHUNKH // 2, body, 0)


_sc_gather = pl.kernel(
    _sc_gather_body,
    out_type=[jax.ShapeDtypeStruct((EH, HID), _f32),
              jax.ShapeDtypeStruct((EH, HID), _f32)],
    mesh=plsc.VectorSubcoreMesh(core_axis_name="c", subcore_axis_name="s"),
    scratch_types=[pltpu.VMEM((NCHUNKH, CHUNK), jnp.int32),
                   pltpu.VMEM((NCHUNKH, CHUNK), jnp.int32),
                   pltpu.VMEM((CHUNK, HID), _f32),
                   pltpu.VMEM((CHUNK, HID), _f32),
                   pltpu.VMEM((CHUNK, HID), _f32),
                   pltpu.VMEM((CHUNK, HID), _f32),
                   pltpu.SemaphoreType.DMA,
                   pltpu.SemaphoreType.DMA,
                   pltpu.SemaphoreType.DMA,
                   pltpu.SemaphoreType.DMA],
)


def _acc_zero(idx_v, buf_v, acc, sid):
    for z, nz in enumerate(NZS):
        off = sid * NPT + z * 128

        def idxrow(i, carry):
            idx_v[pl.ds(i * 16, 16)] = lax.iota(jnp.int32, 16) + (off + i * 16)
            return carry

        lax.fori_loop(0, CHUNK // 16, idxrow, 0)
        pltpu.sync_copy(buf_v.at[pl.ds(0, nz)], acc.at[idx_v.at[pl.ds(0, nz)]])


def _acc_readback(idx_v, buf_v, acc, out_hbm, cid, sid, sem):
    for z, nz in enumerate(NZS):
        off = sid * NPT + z * 128
        foff = cid * NPADN + off

        def idxrow(i, carry):
            idx_v[pl.ds(i * 16, 16)] = lax.iota(jnp.int32, 16) + (off + i * 16)
            return carry

        lax.fori_loop(0, CHUNK // 16, idxrow, 0)
        pltpu.async_copy(acc.at[idx_v.at[pl.ds(0, nz)]],
                         buf_v.at[pl.ds(0, nz)], sem).wait()
        pltpu.sync_copy(buf_v.at[pl.ds(0, nz)], out_hbm.at[pl.ds(foff, nz)])


def _sc_num_body(scl_hbm, sch_hbm, d2l_hbm, d2h_hbm, num_out,
                 zi_v, i2_v, s0, s1, num_acc, semi0, semi1, sems0, sems1):
    cid = lax.axis_index("c")
    sid = lax.axis_index("s")
    wid = cid * NS + sid
    tb = wid * EPWH

    def zrow(i, carry):
        for k in range(HID // 16):
            s0[i, pl.ds(k * 16, 16)] = jnp.zeros((16,), _f32)
        return carry

    lax.fori_loop(0, CHUNK, zrow, 0)
    _acc_zero(zi_v, s0, num_acc, sid)
    plsc.subcore_barrier()

    def run_half(sc_hbm, d2_hbm):
        def starti(j, p, sem):
            pltpu.async_copy(d2_hbm.at[wid * NCHUNKH + j], i2_v.at[p], sem)

        def draini(p, sem):
            pltpu.make_async_copy(d2_hbm.at[0], i2_v.at[p], sem).wait()

        def starts(j, buf, sem):
            pltpu.async_copy(sc_hbm.at[pl.ds(tb + j * CHUNK, CHUNK)], buf, sem)

        def drains(buf, sem):
            pltpu.make_async_copy(sc_hbm.at[pl.ds(0, CHUNK)], buf, sem).wait()

        starti(0, 0, semi0)
        starts(0, s0, sems0)
        starti(1, 1, semi1)
        starts(1, s1, sems1)

        def body(t, carry):
            j0 = 2 * t
            j1 = j0 + 1
            draini(0, semi0)
            drains(s0, sems0)
            pltpu.sync_copy(s0, num_acc.at[i2_v.at[0]], add=True)

            @pl.when(j0 + 2 < NCHUNKH)
            def _():
                starti(j0 + 2, 0, semi0)
                starts(j0 + 2, s0, sems0)

            draini(1, semi1)
            drains(s1, sems1)
            pltpu.sync_copy(s1, num_acc.at[i2_v.at[1]], add=True)

            @pl.when(j1 + 2 < NCHUNKH)
            def _():
                starti(j1 + 2, 1, semi1)
                starts(j1 + 2, s1, sems1)

            return carry

        lax.fori_loop(0, NCHUNKH // 2, body, 0)

    run_half(scl_hbm, d2l_hbm)
    run_half(sch_hbm, d2h_hbm)
    plsc.subcore_barrier()
    _acc_readback(zi_v, s0, num_acc, num_out, cid, sid, sems0)


_sc_num = pl.kernel(
    _sc_num_body,
    out_type=jax.ShapeDtypeStruct((NC * NPADN, HID), _f32),
    mesh=plsc.VectorSubcoreMesh(core_axis_name="c", subcore_axis_name="s"),
    scratch_types=[pltpu.VMEM((CHUNK,), jnp.int32),
                   pltpu.VMEM((2, CHUNK), jnp.int32),
                   pltpu.VMEM((CHUNK, HID), _f32),
                   pltpu.VMEM((CHUNK, HID), _f32),
                   pltpu.VMEM_SHARED((NPADN, HID), _f32),
                   pltpu.SemaphoreType.DMA,
                   pltpu.SemaphoreType.DMA,
                   pltpu.SemaphoreType.DMA,
                   pltpu.SemaphoreType.DMA],
)


def _sc_den_body(exl_hbm, exh_hbm, d2l_hbm, d2h_hbm, den_out,
                 zi_v, i2_v, e0, e1, dbuf_v, den_acc,
                 semi0, semi1, seme0, seme1):
    cid = lax.axis_index("c")
    sid = lax.axis_index("s")
    wid = cid * NS + sid
    tb = wid * EPWH

    def zrow(i, carry):
        for k in range(HID // 16):
            dbuf_v[i, pl.ds(k * 16, 16)] = jnp.zeros((16,), _f32)
        return carry

    lax.fori_loop(0, CHUNK, zrow, 0)
    _acc_zero(zi_v, dbuf_v, den_acc, sid)
    plsc.subcore_barrier()

    def run_half(ex_hbm, d2_hbm):
        def starti(j, p, sem):
            pltpu.async_copy(d2_hbm.at[wid * NCHUNKH + j], i2_v.at[p], sem)

        def draini(p, sem):
            pltpu.make_async_copy(d2_hbm.at[0], i2_v.at[p], sem).wait()

        def starte(j, buf, sem):
            pltpu.async_copy(ex_hbm.at[pl.ds(tb + j * CHUNK, CHUNK)], buf, sem)

        def draine(buf, sem):
            pltpu.make_async_copy(ex_hbm.at[pl.ds(0, CHUNK)], buf, sem).wait()

        def stage_and_scatter(ebuf, p):
            def edge(e, cc):
                dbuf_v[e, pl.ds(0, 16)] = ebuf[e, :]
                return cc

            lax.fori_loop(0, CHUNK, edge, 0)
            pltpu.sync_copy(dbuf_v, den_acc.at[i2_v.at[p]], add=True)

        starti(0, 0, semi0)
        starte(0, e0, seme0)
        starti(1, 1, semi1)
        starte(1, e1, seme1)

        def body(t, carry):
            j0 = 2 * t
            j1 = j0 + 1
            draini(0, semi0)
            draine(e0, seme0)
            stage_and_scatter(e0, 0)

            @pl.when(j0 + 2 < NCHUNKH)
            def _():
                starti(j0 + 2, 0, semi0)
                starte(j0 + 2, e0, seme0)

            draini(1, semi1)
            draine(e1, seme1)
            stage_and_scatter(e1, 1)

            @pl.when(j1 + 2 < NCHUNKH)
            def _():
                starti(j1 + 2, 1, semi1)
                starte(j1 + 2, e1, seme1)

            return carry

        lax.fori_loop(0, NCHUNKH // 2, body, 0)

    run_half(exl_hbm, d2l_hbm)
    run_half(exh_hbm, d2h_hbm)
    plsc.subcore_barrier()
    _acc_readback(zi_v, dbuf_v, den_acc, den_out, cid, sid, seme0)


_sc_den = pl.kernel(
    _sc_den_body,
    out_type=jax.ShapeDtypeStruct((NC * NPADN, HID), _f32),
    mesh=plsc.VectorSubcoreMesh(core_axis_name="c", subcore_axis_name="s"),
    scratch_types=[pltpu.VMEM((CHUNK,), jnp.int32),
                   pltpu.VMEM((2, CHUNK), jnp.int32),
                   pltpu.VMEM((CHUNK, 16), _f32),
                   pltpu.VMEM((CHUNK, 16), _f32),
                   pltpu.VMEM((CHUNK, HID), _f32),
                   pltpu.VMEM_SHARED((NPADN, HID), _f32),
                   pltpu.SemaphoreType.DMA,
                   pltpu.SemaphoreType.DMA,
                   pltpu.SemaphoreType.DMA,
                   pltpu.SemaphoreType.DMA],
)


def _tc_dense1_body(x_ref, wl_ref, wr_ref, xl_out, xr_out):
    x = x_ref[...]
    xl_out[...] = jnp.dot(x, wl_ref[...], preferred_element_type=_f32)
    xr_out[...] = jnp.dot(x, wr_ref[...], preferred_element_type=_f32)


_tc_dense1 = pl.pallas_call(
    _tc_dense1_body,
    out_shape=[jax.ShapeDtypeStruct((N, HID), _f32),
               jax.ShapeDtypeStruct((N, HID), _f32)],
)


def _make_tc_edge(base):
    def _tc_edge_body(xls_ref, xrd_ref, ea_ref, we_ref, a_ref, pexp_ref,
                      sc_out, ex_out):
        bid = pl.program_id(0)
        xls = xls_ref[...]
        m = xls + xrd_ref[...] + jnp.dot(
            ea_ref[...], we_ref[...], preferred_element_type=_f32)
        m = jnp.where(m >= 0, m, 0.2 * m)
        alpha = jnp.dot(m, a_ref[...], preferred_element_type=_f32)
        ex = jnp.exp(alpha)
        col = lax.broadcasted_iota(jnp.int32, (BE, 16), 1) < H
        row = (lax.broadcasted_iota(jnp.int32, (BE, 16), 0)
               + (bid * BE + base)) < E
        ex = jnp.where(col & row, ex, 0.0)
        ex_out[...] = ex
        sc_out[...] = xls * jnp.dot(ex, pexp_ref[...],
                                    preferred_element_type=_f32)

    return pl.pallas_call(
        _tc_edge_body,
        grid=(EGRIDH,),
        in_specs=[pl.BlockSpec((BE, HID), lambda i: (i, 0)),
                  pl.BlockSpec((BE, HID), lambda i: (i, 0)),
                  pl.BlockSpec((BE, ED), lambda i: (i, 0)),
                  pl.BlockSpec((ED, HID), lambda i: (0, 0)),
                  pl.BlockSpec((HID, 16), lambda i: (0, 0)),
                  pl.BlockSpec((16, HID), lambda i: (0, 0))],
        out_specs=[pl.BlockSpec((BE, HID), lambda i: (i, 0)),
                   pl.BlockSpec((BE, 16), lambda i: (i, 0))],
        out_shape=[jax.ShapeDtypeStruct((EH, HID), _f32),
                   jax.ShapeDtypeStruct((EH, 16), _f32)],
    )


_tc_edge_lo = _make_tc_edge(0)
_tc_edge_hi = _make_tc_edge(EH)


def _post_h(na, nb, da, db, pexp, b):
    d128 = jnp.dot(da + db, pexp, preferred_element_type=_f32)
    hpre = (na + nb) / (d128 + 1e-16) + b
    return jnp.where(hpre > 0, hpre, jnp.exp(jnp.minimum(hpre, 0.0)) - 1.0)


def _tc_dense2_body(na_ref, nb_ref, da_ref, db_ref, pexp_ref, b_ref,
                    wl_ref, wr_ref, xl_out, xr_out):
    h = _post_h(na_ref[...], nb_ref[...], da_ref[...], db_ref[...],
                pexp_ref[...], b_ref[...])
    xl_out[...] = jnp.dot(h, wl_ref[...], preferred_element_type=_f32)
    xr_out[...] = jnp.dot(h, wr_ref[...], preferred_element_type=_f32)


_tc_dense2 = pl.pallas_call(
    _tc_dense2_body,
    out_shape=[jax.ShapeDtypeStruct((N, HID), _f32),
               jax.ShapeDtypeStruct((N, HID), _f32)],
)


def _tc_class_body(na_ref, nb_ref, da_ref, db_ref, pexp_ref, b_ref,
                   wc_ref, bc_ref, out_ref):
    h = _post_h(na_ref[...], nb_ref[...], da_ref[...], db_ref[...],
                pexp_ref[...], b_ref[...])
    out_ref[...] = jnp.dot(h, wc_ref[...], preferred_element_type=_f32) + bc_ref[...]


_tc_class = pl.pallas_call(
    _tc_class_body,
    out_shape=jax.ShapeDtypeStruct((N, OUT), _f32),
)


def _make_att_blockdiag(att):
    eye8 = jnp.eye(H, dtype=_f32)
    a = (att[:, :, None] * eye8[:, None, :]).reshape(HID, H)
    return jnp.pad(a, ((0, 0), (0, 16 - H)))


def _make_head_expand():
    eye8 = jnp.eye(H, dtype=_f32)
    p = (eye8[:, :, None] * jnp.ones((1, 1, C), _f32)).reshape(H, HID)
    return jnp.pad(p, ((0, 16 - H), (0, 0)))


def _layer(xl, xr, prep, We, a, pexp):
    s2l, s2h, d2l, d2h, eal, eah = prep
    xlsl, xrdl = _sc_gather(xl, xr, s2l, d2l)
    scl, exl = _tc_edge_lo(xlsl, xrdl, eal, We, a, pexp)
    xlsh, xrdh = _sc_gather(xl, xr, s2h, d2h)
    sch, exh = _tc_edge_hi(xlsh, xrdh, eah, We, a, pexp)
    num = _sc_num(scl, sch, d2l, d2h).reshape(NC, NPADN, HID)
    den = _sc_den(exl, exh, d2l, d2h).reshape(NC, NPADN, HID)[:, :, :16]
    return num, den


def kernel(x, edge_index, edge_attr, Wl1, Wr1, We1, att1, b1,
           Wl2, Wr2, We2, att2, b2, Wc, bc):
    srcp = jnp.pad(edge_index[0], (0, EPAD - E))
    dstp = jnp.pad(edge_index[1], (0, EPAD - E))
    src2 = srcp.reshape(EPAD // CHUNK, CHUNK)
    dst2 = dstp.reshape(EPAD // CHUNK, CHUNK)
    eap = jnp.pad(edge_attr, ((0, EPAD - E), (0, 0)))
    nrh = EH // CHUNK
    prep = (src2[:nrh], src2[nrh:], dst2[:nrh], dst2[nrh:],
            eap[:EH], eap[EH:])
    pexp = _make_head_expand()
    a1 = _make_att_blockdiag(att1)
    a2 = _make_att_blockdiag(att2)
    b1r = b1.reshape(1, HID)
    b2r = b2.reshape(1, HID)
    bcr = bc.reshape(1, OUT)

    xl1, xr1 = _tc_dense1(x, Wl1, Wr1)
    num1, den1 = _layer(xl1, xr1, prep, We1, a1, pexp)
    xl2, xr2 = _tc_dense2(num1[0, :N], num1[1, :N], den1[0, :N], den1[1, :N],
                          pexp, b1r, Wl2, Wr2)
    num2, den2 = _layer(xl2, xr2, prep, We2, a2, pexp)
    return _tc_class(num2[0, :N], num2[1, :N], den2[0, :N], den2[1, :N],
                     pexp, b2r, Wc, bcr)


# speedup vs baseline: 1.0596x; 1.0596x over previous
import functools

import jax
import jax.numpy as jnp
from jax import lax
from jax.experimental import pallas as pl
from jax.experimental.pallas import tpu as pltpu
from jax.experimental.pallas import tpu_sc as plsc

N = 10000
E = 320000
IN = 128
ED = 16
H = 8
C = 16
HID = H * C
OUT = 40

NC = 2
NS = 16
NW = NC * NS
EPAD = 327680
EPWH = EPAD // NW
CHUNK = 128
NCHUNKH = EPWH // CHUNK
NPADN = 10112
NPT = NPADN // NS
NZS = (128, 128, 128, 128, 120)

BE = 2560
EGRIDH = EPAD // BE

_f32 = jnp.float32


def _sc_gather_body(xl_hbm, xr_hbm, src2_hbm, dst2_hbm, xls_out, xrd_out,
                    idxs_v, idxd_v, a0, a1, b0, b1, sa0, sa1, sb0, sb1):
    cid = lax.axis_index("c")
    sid = lax.axis_index("s")
    wid = cid * NS + sid
    tb = wid * EPWH

    pltpu.sync_copy(src2_hbm.at[pl.ds(wid * NCHUNKH, NCHUNKH)], idxs_v)
    pltpu.sync_copy(dst2_hbm.at[pl.ds(wid * NCHUNKH, NCHUNKH)], idxd_v)

    def start(j, buf, sem, table, idx):
        pltpu.async_copy(table.at[idx.at[j]], buf, sem)

    def drain(buf, sem, table, idx):
        pltpu.make_async_copy(table.at[idx.at[0]], buf, sem).wait()

    start(0, a0, sa0, xl_hbm, idxs_v)
    start(0, b0, sb0, xr_hbm, idxd_v)
    start(1, a1, sa1, xl_hbm, idxs_v)
    start(1, b1, sb1, xr_hbm, idxd_v)

    def body(t, carry):
        j0 = 2 * t
        j1 = j0 + 1
        drain(a0, sa0, xl_hbm, idxs_v)
        pltpu.sync_copy(a0, xls_out.at[pl.ds(tb + j0 * CHUNK, CHUNK)])
        drain(b0, sb0, xr_hbm, idxd_v)
        pltpu.sync_copy(b0, xrd_out.at[pl.ds(tb + j0 * CHUNK, CHUNK)])

        @pl.when(j0 + 2 < NCHUNKH)
        def _():
            start(j0 + 2, a0, sa0, xl_hbm, idxs_v)
            start(j0 + 2, b0, sb0, xr_hbm, idxd_v)

        drain(a1, sa1, xl_hbm, idxs_v)
        pltpu.sync_copy(a1, xls_out.at[pl.ds(tb + j1 * CHUNK, CHUNK)])
        drain(b1, sb1, xr_hbm, idxd_v)
        pltpu.sync_copy(b1, xrd_out.at[pl.ds(tb + j1 * CHUNK, CHUNK)])

        @pl.when(j1 + 2 < NCHUNKH)
        def _():
            start(j1 + 2, a1, sa1, xl_hbm, idxs_v)
            start(j1 + 2, b1, sb1, xr_hbm, idxd_v)

        return carry

    lax.fori_loop(0, NCHUNKH // 2, body, 0)


_sc_gather = pl.kernel(
    _sc_gather_body,
    out_type=[jax.ShapeDtypeStruct((EPAD, HID), _f32),
              jax.ShapeDtypeStruct((EPAD, HID), _f32)],
    mesh=plsc.VectorSubcoreMesh(core_axis_name="c", subcore_axis_name="s"),
    scratch_types=[pltpu.VMEM((NCHUNKH, CHUNK), jnp.int32),
                   pltpu.VMEM((NCHUNKH, CHUNK), jnp.int32),
                   pltpu.VMEM((CHUNK, HID), _f32),
                   pltpu.VMEM((CHUNK, HID), _f32),
                   pltpu.VMEM((CHUNK, HID), _f32),
                   pltpu.VMEM((CHUNK, HID), _f32),
                   pltpu.SemaphoreType.DMA,
                   pltpu.SemaphoreType.DMA,
                   pltpu.SemaphoreType.DMA,
                   pltpu.SemaphoreType.DMA],
)


def _acc_zero(idx_v, buf_v, acc, sid):
    for z, nz in enumerate(NZS):
        off = sid * NPT + z * 128

        def idxrow(i, carry):
            idx_v[pl.ds(i * 16, 16)] = lax.iota(jnp.int32, 16) + (off + i * 16)
            return carry

        lax.fori_loop(0, CHUNK // 16, idxrow, 0)
        pltpu.sync_copy(buf_v.at[pl.ds(0, nz)], acc.at[idx_v.at[pl.ds(0, nz)]])


def _acc_readback(idx_v, buf_v, acc, out_hbm, cid, sid, sem):
    for z, nz in enumerate(NZS):
        off = sid * NPT + z * 128
        foff = cid * NPADN + off

        def idxrow(i, carry):
            idx_v[pl.ds(i * 16, 16)] = lax.iota(jnp.int32, 16) + (off + i * 16)
            return carry

        lax.fori_loop(0, CHUNK // 16, idxrow, 0)
        pltpu.async_copy(acc.at[idx_v.at[pl.ds(0, nz)]],
                         buf_v.at[pl.ds(0, nz)], sem).wait()
        pltpu.sync_copy(buf_v.at[pl.ds(0, nz)], out_hbm.at[pl.ds(foff, nz)])


def _sc_num_body(sc_hbm, d2_hbm, num_out,
                 zi_v, i2_v, s0, s1, num_acc, semi0, semi1, sems0, sems1):
    cid = lax.axis_index("c")
    sid = lax.axis_index("s")
    wid = cid * NS + sid
    tb = wid * EPWH

    def zrow(i, carry):
        for k in range(HID // 16):
            s0[i, pl.ds(k * 16, 16)] = jnp.zeros((16,), _f32)
        return carry

    lax.fori_loop(0, CHUNK, zrow, 0)
    _acc_zero(zi_v, s0, num_acc, sid)
    plsc.subcore_barrier()

    def run_half(sc_hbm, d2_hbm):
        def starti(j, p, sem):
            pltpu.async_copy(d2_hbm.at[wid * NCHUNKH + j], i2_v.at[p], sem)

        def draini(p, sem):
            pltpu.make_async_copy(d2_hbm.at[0], i2_v.at[p], sem).wait()

        def starts(j, buf, sem):
            pltpu.async_copy(sc_hbm.at[pl.ds(tb + j * CHUNK, CHUNK)], buf, sem)

        def drains(buf, sem):
            pltpu.make_async_copy(sc_hbm.at[pl.ds(0, CHUNK)], buf, sem).wait()

        starti(0, 0, semi0)
        starts(0, s0, sems0)
        starti(1, 1, semi1)
        starts(1, s1, sems1)

        def body(t, carry):
            j0 = 2 * t
            j1 = j0 + 1
            draini(0, semi0)
            drains(s0, sems0)
            pltpu.sync_copy(s0, num_acc.at[i2_v.at[0]], add=True)

            @pl.when(j0 + 2 < NCHUNKH)
            def _():
                starti(j0 + 2, 0, semi0)
                starts(j0 + 2, s0, sems0)

            draini(1, semi1)
            drains(s1, sems1)
            pltpu.sync_copy(s1, num_acc.at[i2_v.at[1]], add=True)

            @pl.when(j1 + 2 < NCHUNKH)
            def _():
                starti(j1 + 2, 1, semi1)
                starts(j1 + 2, s1, sems1)

            return carry

        lax.fori_loop(0, NCHUNKH // 2, body, 0)

    run_half(sc_hbm, d2_hbm)
    plsc.subcore_barrier()
    _acc_readback(zi_v, s0, num_acc, num_out, cid, sid, sems0)


_sc_num = pl.kernel(
    _sc_num_body,
    out_type=jax.ShapeDtypeStruct((NC * NPADN, HID), _f32),
    mesh=plsc.VectorSubcoreMesh(core_axis_name="c", subcore_axis_name="s"),
    scratch_types=[pltpu.VMEM((CHUNK,), jnp.int32),
                   pltpu.VMEM((2, CHUNK), jnp.int32),
                   pltpu.VMEM((CHUNK, HID), _f32),
                   pltpu.VMEM((CHUNK, HID), _f32),
                   pltpu.VMEM_SHARED((NPADN, HID), _f32),
                   pltpu.SemaphoreType.DMA,
                   pltpu.SemaphoreType.DMA,
                   pltpu.SemaphoreType.DMA,
                   pltpu.SemaphoreType.DMA],
)


def _sc_den_body(ex_hbm, d2_hbm, den_out,
                 zi_v, i2_v, e0, e1, dbuf_v, den_acc,
                 semi0, semi1, seme0, seme1):
    cid = lax.axis_index("c")
    sid = lax.axis_index("s")
    wid = cid * NS + sid
    tb = wid * EPWH

    def zrow(i, carry):
        for k in range(HID // 16):
            dbuf_v[i, pl.ds(k * 16, 16)] = jnp.zeros((16,), _f32)
        return carry

    lax.fori_loop(0, CHUNK, zrow, 0)
    _acc_zero(zi_v, dbuf_v, den_acc, sid)
    plsc.subcore_barrier()

    def run_half(exr_hbm, d2r_hbm):
        def starti(j, p, sem):
            pltpu.async_copy(d2r_hbm.at[wid * NCHUNKH + j], i2_v.at[p], sem)

        def draini(p, sem):
            pltpu.make_async_copy(d2r_hbm.at[0], i2_v.at[p], sem).wait()

        def starte(j, buf, sem):
            pltpu.async_copy(exr_hbm.at[pl.ds(tb + j * CHUNK, CHUNK)], buf, sem)

        def draine(buf, sem):
            pltpu.make_async_copy(exr_hbm.at[pl.ds(0, CHUNK)], buf, sem).wait()

        def stage_and_scatter(ebuf, p):
            def edge(e, cc):
                dbuf_v[e, pl.ds(0, 16)] = ebuf[e, :]
                return cc

            lax.fori_loop(0, CHUNK, edge, 0)
            pltpu.sync_copy(dbuf_v, den_acc.at[i2_v.at[p]], add=True)

        starti(0, 0, semi0)
        starte(0, e0, seme0)
        starti(1, 1, semi1)
        starte(1, e1, seme1)

        def body(t, carry):
            j0 = 2 * t
            j1 = j0 + 1
            draini(0, semi0)
            draine(e0, seme0)
            stage_and_scatter(e0, 0)

            @pl.when(j0 + 2 < NCHUNKH)
            def _():
                starti(j0 + 2, 0, semi0)
                starte(j0 + 2, e0, seme0)

            draini(1, semi1)
            draine(e1, seme1)
            stage_and_scatter(e1, 1)

            @pl.when(j1 + 2 < NCHUNKH)
            def _():
                starti(j1 + 2, 1, semi1)
                starte(j1 + 2, e1, seme1)

            return carry

        lax.fori_loop(0, NCHUNKH // 2, body, 0)

    run_half(ex_hbm, d2_hbm)
    plsc.subcore_barrier()
    _acc_readback(zi_v, dbuf_v, den_acc, den_out, cid, sid, seme0)


_sc_den = pl.kernel(
    _sc_den_body,
    out_type=jax.ShapeDtypeStruct((NC * NPADN, HID), _f32),
    mesh=plsc.VectorSubcoreMesh(core_axis_name="c", subcore_axis_name="s"),
    scratch_types=[pltpu.VMEM((CHUNK,), jnp.int32),
                   pltpu.VMEM((2, CHUNK), jnp.int32),
                   pltpu.VMEM((CHUNK, 16), _f32),
                   pltpu.VMEM((CHUNK, 16), _f32),
                   pltpu.VMEM((CHUNK, HID), _f32),
                   pltpu.VMEM_SHARED((NPADN, HID), _f32),
                   pltpu.SemaphoreType.DMA,
                   pltpu.SemaphoreType.DMA,
                   pltpu.SemaphoreType.DMA,
                   pltpu.SemaphoreType.DMA],
)


def _tc_dense1_body(x_ref, wl_ref, wr_ref, xl_out, xr_out):
    x = x_ref[...]
    xl_out[...] = jnp.dot(x, wl_ref[...], preferred_element_type=_f32)
    xr_out[...] = jnp.dot(x, wr_ref[...], preferred_element_type=_f32)


_tc_dense1 = pl.pallas_call(
    _tc_dense1_body,
    out_shape=[jax.ShapeDtypeStruct((N, HID), _f32),
               jax.ShapeDtypeStruct((N, HID), _f32)],
)


def _tc_edge_body(xls_ref, xrd_ref, ea_ref, we_ref, a_ref, pexp_ref,
                  sc_out, ex_out):
    bid = pl.program_id(0)
    xls = xls_ref[...]
    m = xls + xrd_ref[...] + jnp.dot(
        ea_ref[...], we_ref[...], preferred_element_type=_f32)
    m = jnp.where(m >= 0, m, 0.2 * m)
    alpha = jnp.dot(m, a_ref[...], preferred_element_type=_f32)
    ex = jnp.exp(alpha)
    col = lax.broadcasted_iota(jnp.int32, (BE, 16), 1) < H
    row = (lax.broadcasted_iota(jnp.int32, (BE, 16), 0) + bid * BE) < E
    ex = jnp.where(col & row, ex, 0.0)
    ex_out[...] = ex
    sc_out[...] = xls * jnp.dot(ex, pexp_ref[...], preferred_element_type=_f32)


_tc_edge = pl.pallas_call(
    _tc_edge_body,
    grid=(EGRIDH,),
    in_specs=[pl.BlockSpec((BE, HID), lambda i: (i, 0)),
              pl.BlockSpec((BE, HID), lambda i: (i, 0)),
              pl.BlockSpec((BE, ED), lambda i: (i, 0)),
              pl.BlockSpec((ED, HID), lambda i: (0, 0)),
              pl.BlockSpec((HID, 16), lambda i: (0, 0)),
              pl.BlockSpec((16, HID), lambda i: (0, 0))],
    out_specs=[pl.BlockSpec((BE, HID), lambda i: (i, 0)),
               pl.BlockSpec((BE, 16), lambda i: (i, 0))],
    out_shape=[jax.ShapeDtypeStruct((EPAD, HID), _f32),
               jax.ShapeDtypeStruct((EPAD, 16), _f32)],
)


def _post_h(na, nb, da, db, pexp, b):
    d128 = jnp.dot(da + db, pexp, preferred_element_type=_f32)
    hpre = (na + nb) / (d128 + 1e-16) + b
    return jnp.where(hpre > 0, hpre, jnp.exp(jnp.minimum(hpre, 0.0)) - 1.0)


def _tc_dense2_body(na_ref, nb_ref, da_ref, db_ref, pexp_ref, b_ref,
                    wl_ref, wr_ref, xl_out, xr_out):
    h = _post_h(na_ref[...], nb_ref[...], da_ref[...], db_ref[...],
                pexp_ref[...], b_ref[...])
    xl_out[...] = jnp.dot(h, wl_ref[...], preferred_element_type=_f32)
    xr_out[...] = jnp.dot(h, wr_ref[...], preferred_element_type=_f32)


_tc_dense2 = pl.pallas_call(
    _tc_dense2_body,
    out_shape=[jax.ShapeDtypeStruct((N, HID), _f32),
               jax.ShapeDtypeStruct((N, HID), _f32)],
)


def _tc_class_body(na_ref, nb_ref, da_ref, db_ref, pexp_ref, b_ref,
                   wc_ref, bc_ref, out_ref):
    h = _post_h(na_ref[...], nb_ref[...], da_ref[...], db_ref[...],
                pexp_ref[...], b_ref[...])
    out_ref[...] = jnp.dot(h, wc_ref[...], preferred_element_type=_f32) + bc_ref[...]


_tc_class = pl.pallas_call(
    _tc_class_body,
    out_shape=jax.ShapeDtypeStruct((N, OUT), _f32),
)


def _make_att_blockdiag(att):
    eye8 = jnp.eye(H, dtype=_f32)
    a = (att[:, :, None] * eye8[:, None, :]).reshape(HID, H)
    return jnp.pad(a, ((0, 0), (0, 16 - H)))


def _make_head_expand():
    eye8 = jnp.eye(H, dtype=_f32)
    p = (eye8[:, :, None] * jnp.ones((1, 1, C), _f32)).reshape(H, HID)
    return jnp.pad(p, ((0, 16 - H), (0, 0)))


def _layer(xl, xr, prep, We, a, pexp):
    src2, dst2, eap = prep
    xls, xrd = _sc_gather(xl, xr, src2, dst2)
    sc, ex = _tc_edge(xls, xrd, eap, We, a, pexp)
    num = _sc_num(sc, dst2).reshape(NC, NPADN, HID)
    den = _sc_den(ex, dst2).reshape(NC, NPADN, HID)[:, :, :16]
    return num, den


def kernel(x, edge_index, edge_attr, Wl1, Wr1, We1, att1, b1,
           Wl2, Wr2, We2, att2, b2, Wc, bc):
    srcp = jnp.pad(edge_index[0], (0, EPAD - E))
    dstp = jnp.pad(edge_index[1], (0, EPAD - E))
    src2 = srcp.reshape(EPAD // CHUNK, CHUNK)
    dst2 = dstp.reshape(EPAD // CHUNK, CHUNK)
    eap = jnp.pad(edge_attr, ((0, EPAD - E), (0, 0)))
    prep = (src2, dst2, eap)
    pexp = _make_head_expand()
    a1 = _make_att_blockdiag(att1)
    a2 = _make_att_blockdiag(att2)
    b1r = b1.reshape(1, HID)
    b2r = b2.reshape(1, HID)
    bcr = bc.reshape(1, OUT)

    xl1, xr1 = _tc_dense1(x, Wl1, Wr1)
    num1, den1 = _layer(xl1, xr1, prep, We1, a1, pexp)
    xl2, xr2 = _tc_dense2(num1[0, :N], num1[1, :N], den1[0, :N], den1[1, :N],
                          pexp, b1r, Wl2, Wr2)
    num2, den2 = _layer(xl2, xr2, prep, We2, a2, pexp)
    return _tc_class(num2[0, :N], num2[1, :N], den2[0, :N], den2[1, :N],
                     pexp, b2r, Wc, bcr)


# speedup vs baseline: 1.0699x; 1.0097x over previous
import jax
import jax.numpy as jnp
from jax import lax
from jax.experimental import pallas as pl
from jax.experimental.pallas import tpu as pltpu
from jax.experimental.pallas import tpu_sc as plsc

N = 10000
E = 320000
IN = 128
ED = 16
H = 8
C = 16
HID = H * C
OUT = 40

NC = 2
NS = 16
NW = NC * NS
EPAD = 327680
EPWH = EPAD // NW
CHUNK = 128
NCHUNKH = EPWH // CHUNK
NPADN = 10112
NPT = NPADN // NS
NZS = (128, 128, 128, 128, 120)

BE = 5120
EGRIDH = EPAD // BE

_f32 = jnp.float32


def _sc_gather_body(xl_hbm, xr_hbm, src2_hbm, dst2_hbm, xls_out, xrd_out,
                    idxs_v, idxd_v, a0, a1, b0, b1, sa0, sa1, sb0, sb1):
    cid = lax.axis_index("c")
    sid = lax.axis_index("s")
    wid = cid * NS + sid
    tb = wid * EPWH

    pltpu.sync_copy(src2_hbm.at[pl.ds(wid * NCHUNKH, NCHUNKH)], idxs_v)
    pltpu.sync_copy(dst2_hbm.at[pl.ds(wid * NCHUNKH, NCHUNKH)], idxd_v)

    def start(j, buf, sem, table, idx):
        pltpu.async_copy(table.at[idx.at[j]], buf, sem)

    def drain(buf, sem, table, idx):
        pltpu.make_async_copy(table.at[idx.at[0]], buf, sem).wait()

    start(0, a0, sa0, xl_hbm, idxs_v)
    start(0, b0, sb0, xr_hbm, idxd_v)
    start(1, a1, sa1, xl_hbm, idxs_v)
    start(1, b1, sb1, xr_hbm, idxd_v)

    def body(t, carry):
        j0 = 2 * t
        j1 = j0 + 1
        drain(a0, sa0, xl_hbm, idxs_v)
        pltpu.sync_copy(a0, xls_out.at[pl.ds(tb + j0 * CHUNK, CHUNK)])
        drain(b0, sb0, xr_hbm, idxd_v)
        pltpu.sync_copy(b0, xrd_out.at[pl.ds(tb + j0 * CHUNK, CHUNK)])

        @pl.when(j0 + 2 < NCHUNKH)
        def _():
            start(j0 + 2, a0, sa0, xl_hbm, idxs_v)
            start(j0 + 2, b0, sb0, xr_hbm, idxd_v)

        drain(a1, sa1, xl_hbm, idxs_v)
        pltpu.sync_copy(a1, xls_out.at[pl.ds(tb + j1 * CHUNK, CHUNK)])
        drain(b1, sb1, xr_hbm, idxd_v)
        pltpu.sync_copy(b1, xrd_out.at[pl.ds(tb + j1 * CHUNK, CHUNK)])

        @pl.when(j1 + 2 < NCHUNKH)
        def _():
            start(j1 + 2, a1, sa1, xl_hbm, idxs_v)
            start(j1 + 2, b1, sb1, xr_hbm, idxd_v)

        return carry

    lax.fori_loop(0, NCHUNKH // 2, body, 0)


_sc_gather = pl.kernel(
    _sc_gather_body,
    out_type=[jax.ShapeDtypeStruct((EPAD, HID), _f32),
              jax.ShapeDtypeStruct((EPAD, HID), _f32)],
    mesh=plsc.VectorSubcoreMesh(core_axis_name="c", subcore_axis_name="s"),
    scratch_types=[pltpu.VMEM((NCHUNKH, CHUNK), jnp.int32),
                   pltpu.VMEM((NCHUNKH, CHUNK), jnp.int32),
                   pltpu.VMEM((CHUNK, HID), _f32),
                   pltpu.VMEM((CHUNK, HID), _f32),
                   pltpu.VMEM((CHUNK, HID), _f32),
                   pltpu.VMEM((CHUNK, HID), _f32),
                   pltpu.SemaphoreType.DMA,
                   pltpu.SemaphoreType.DMA,
                   pltpu.SemaphoreType.DMA,
                   pltpu.SemaphoreType.DMA],
)


def _acc_zero(idx_v, buf_v, acc, sid):
    for z, nz in enumerate(NZS):
        off = sid * NPT + z * 128

        def idxrow(i, carry):
            idx_v[pl.ds(i * 16, 16)] = lax.iota(jnp.int32, 16) + (off + i * 16)
            return carry

        lax.fori_loop(0, CHUNK // 16, idxrow, 0)
        pltpu.sync_copy(buf_v.at[pl.ds(0, nz)], acc.at[idx_v.at[pl.ds(0, nz)]])


def _acc_readback(idx_v, buf_v, acc, out_hbm, cid, sid, sem):
    for z, nz in enumerate(NZS):
        off = sid * NPT + z * 128
        foff = cid * NPADN + off

        def idxrow(i, carry):
            idx_v[pl.ds(i * 16, 16)] = lax.iota(jnp.int32, 16) + (off + i * 16)
            return carry

        lax.fori_loop(0, CHUNK // 16, idxrow, 0)
        pltpu.async_copy(acc.at[idx_v.at[pl.ds(0, nz)]],
                         buf_v.at[pl.ds(0, nz)], sem).wait()
        pltpu.sync_copy(buf_v.at[pl.ds(0, nz)], out_hbm.at[pl.ds(foff, nz)])


def _sc_num_body(sc_hbm, d2_hbm, num_out,
                 zi_v, i2_v, s0, s1, num_acc, semi0, semi1, sems0, sems1):
    cid = lax.axis_index("c")
    sid = lax.axis_index("s")
    wid = cid * NS + sid
    tb = wid * EPWH

    def zrow(i, carry):
        for k in range(HID // 16):
            s0[i, pl.ds(k * 16, 16)] = jnp.zeros((16,), _f32)
        return carry

    lax.fori_loop(0, CHUNK, zrow, 0)
    _acc_zero(zi_v, s0, num_acc, sid)
    plsc.subcore_barrier()

    def run_half(sc_hbm, d2_hbm):
        def starti(j, p, sem):
            pltpu.async_copy(d2_hbm.at[wid * NCHUNKH + j], i2_v.at[p], sem)

        def draini(p, sem):
            pltpu.make_async_copy(d2_hbm.at[0], i2_v.at[p], sem).wait()

        def starts(j, buf, sem):
            pltpu.async_copy(sc_hbm.at[pl.ds(tb + j * CHUNK, CHUNK)], buf, sem)

        def drains(buf, sem):
            pltpu.make_async_copy(sc_hbm.at[pl.ds(0, CHUNK)], buf, sem).wait()

        starti(0, 0, semi0)
        starts(0, s0, sems0)
        starti(1, 1, semi1)
        starts(1, s1, sems1)

        def body(t, carry):
            j0 = 2 * t
            j1 = j0 + 1
            draini(0, semi0)
            drains(s0, sems0)
            pltpu.sync_copy(s0, num_acc.at[i2_v.at[0]], add=True)

            @pl.when(j0 + 2 < NCHUNKH)
            def _():
                starti(j0 + 2, 0, semi0)
                starts(j0 + 2, s0, sems0)

            draini(1, semi1)
            drains(s1, sems1)
            pltpu.sync_copy(s1, num_acc.at[i2_v.at[1]], add=True)

            @pl.when(j1 + 2 < NCHUNKH)
            def _():
                starti(j1 + 2, 1, semi1)
                starts(j1 + 2, s1, sems1)

            return carry

        lax.fori_loop(0, NCHUNKH // 2, body, 0)

    run_half(sc_hbm, d2_hbm)
    plsc.subcore_barrier()
    _acc_readback(zi_v, s0, num_acc, num_out, cid, sid, sems0)


_sc_num = pl.kernel(
    _sc_num_body,
    out_type=jax.ShapeDtypeStruct((NC * NPADN, HID), _f32),
    mesh=plsc.VectorSubcoreMesh(core_axis_name="c", subcore_axis_name="s"),
    scratch_types=[pltpu.VMEM((CHUNK,), jnp.int32),
                   pltpu.VMEM((2, CHUNK), jnp.int32),
                   pltpu.VMEM((CHUNK, HID), _f32),
                   pltpu.VMEM((CHUNK, HID), _f32),
                   pltpu.VMEM_SHARED((NPADN, HID), _f32),
                   pltpu.SemaphoreType.DMA,
                   pltpu.SemaphoreType.DMA,
                   pltpu.SemaphoreType.DMA,
                   pltpu.SemaphoreType.DMA],
)


def _sc_den_body(ex_hbm, d2_hbm, den_out,
                 zi_v, i2_v, e0, e1, dbuf_v, den_acc,
                 semi0, semi1, seme0, seme1):
    cid = lax.axis_index("c")
    sid = lax.axis_index("s")
    wid = cid * NS + sid
    tb = wid * EPWH

    def zrow(i, carry):
        for k in range(HID // 16):
            dbuf_v[i, pl.ds(k * 16, 16)] = jnp.zeros((16,), _f32)
        return carry

    lax.fori_loop(0, CHUNK, zrow, 0)
    _acc_zero(zi_v, dbuf_v, den_acc, sid)
    plsc.subcore_barrier()

    def run_half(exr_hbm, d2r_hbm):
        def starti(j, p, sem):
            pltpu.async_copy(d2r_hbm.at[wid * NCHUNKH + j], i2_v.at[p], sem)

        def draini(p, sem):
            pltpu.make_async_copy(d2r_hbm.at[0], i2_v.at[p], sem).wait()

        def starte(j, buf, sem):
            pltpu.async_copy(exr_hbm.at[pl.ds(tb + j * CHUNK, CHUNK)], buf, sem)

        def draine(buf, sem):
            pltpu.make_async_copy(exr_hbm.at[pl.ds(0, CHUNK)], buf, sem).wait()

        def stage_and_scatter(ebuf, p):
            def edge(e, cc):
                dbuf_v[e, pl.ds(0, 16)] = ebuf[e, :]
                return cc

            lax.fori_loop(0, CHUNK, edge, 0)
            pltpu.sync_copy(dbuf_v, den_acc.at[i2_v.at[p]], add=True)

        starti(0, 0, semi0)
        starte(0, e0, seme0)
        starti(1, 1, semi1)
        starte(1, e1, seme1)

        def body(t, carry):
            j0 = 2 * t
            j1 = j0 + 1
            draini(0, semi0)
            draine(e0, seme0)
            stage_and_scatter(e0, 0)

            @pl.when(j0 + 2 < NCHUNKH)
            def _():
                starti(j0 + 2, 0, semi0)
                starte(j0 + 2, e0, seme0)

            draini(1, semi1)
            draine(e1, seme1)
            stage_and_scatter(e1, 1)

            @pl.when(j1 + 2 < NCHUNKH)
            def _():
                starti(j1 + 2, 1, semi1)
                starte(j1 + 2, e1, seme1)

            return carry

        lax.fori_loop(0, NCHUNKH // 2, body, 0)

    run_half(ex_hbm, d2_hbm)
    plsc.subcore_barrier()
    _acc_readback(zi_v, dbuf_v, den_acc, den_out, cid, sid, seme0)


_sc_den = pl.kernel(
    _sc_den_body,
    out_type=jax.ShapeDtypeStruct((NC * NPADN, HID), _f32),
    mesh=plsc.VectorSubcoreMesh(core_axis_name="c", subcore_axis_name="s"),
    scratch_types=[pltpu.VMEM((CHUNK,), jnp.int32),
                   pltpu.VMEM((2, CHUNK), jnp.int32),
                   pltpu.VMEM((CHUNK, 16), _f32),
                   pltpu.VMEM((CHUNK, 16), _f32),
                   pltpu.VMEM((CHUNK, HID), _f32),
                   pltpu.VMEM_SHARED((NPADN, HID), _f32),
                   pltpu.SemaphoreType.DMA,
                   pltpu.SemaphoreType.DMA,
                   pltpu.SemaphoreType.DMA,
                   pltpu.SemaphoreType.DMA],
)


def _tc_dense1_body(x_ref, wl_ref, wr_ref, xl_out, xr_out):
    x = x_ref[...]
    xl_out[...] = jnp.dot(x, wl_ref[...], preferred_element_type=_f32)
    xr_out[...] = jnp.dot(x, wr_ref[...], preferred_element_type=_f32)


_tc_dense1 = pl.pallas_call(
    _tc_dense1_body,
    out_shape=[jax.ShapeDtypeStruct((N, HID), _f32),
               jax.ShapeDtypeStruct((N, HID), _f32)],
)


def _tc_edge_body(xls_ref, xrd_ref, ea_ref, we_ref, a_ref, pexp_ref,
                  sc_out, ex_out):
    bid = pl.program_id(0)
    xls = xls_ref[...]
    m = xls + xrd_ref[...] + jnp.dot(
        ea_ref[...], we_ref[...], preferred_element_type=_f32)
    m = jnp.where(m >= 0, m, 0.2 * m)
    alpha = jnp.dot(m, a_ref[...], preferred_element_type=_f32)
    ex = jnp.exp(alpha)
    col = lax.broadcasted_iota(jnp.int32, (BE, 16), 1) < H
    row = (lax.broadcasted_iota(jnp.int32, (BE, 16), 0) + bid * BE) < E
    ex = jnp.where(col & row, ex, 0.0)
    ex_out[...] = ex
    sc_out[...] = xls * jnp.dot(ex, pexp_ref[...], preferred_element_type=_f32)


_tc_edge = pl.pallas_call(
    _tc_edge_body,
    grid=(EGRIDH,),
    in_specs=[pl.BlockSpec((BE, HID), lambda i: (i, 0)),
              pl.BlockSpec((BE, HID), lambda i: (i, 0)),
              pl.BlockSpec((BE, ED), lambda i: (i, 0)),
              pl.BlockSpec((ED, HID), lambda i: (0, 0)),
              pl.BlockSpec((HID, 16), lambda i: (0, 0)),
              pl.BlockSpec((16, HID), lambda i: (0, 0))],
    out_specs=[pl.BlockSpec((BE, HID), lambda i: (i, 0)),
               pl.BlockSpec((BE, 16), lambda i: (i, 0))],
    out_shape=[jax.ShapeDtypeStruct((EPAD, HID), _f32),
               jax.ShapeDtypeStruct((EPAD, 16), _f32)],
)


def _post_h(na, nb, da, db, pexp, b):
    d128 = jnp.dot(da + db, pexp, preferred_element_type=_f32)
    hpre = (na + nb) / (d128 + 1e-16) + b
    return jnp.where(hpre > 0, hpre, jnp.exp(jnp.minimum(hpre, 0.0)) - 1.0)


def _tc_dense2_body(na_ref, nb_ref, da_ref, db_ref, pexp_ref, b_ref,
                    wl_ref, wr_ref, xl_out, xr_out):
    h = _post_h(na_ref[...], nb_ref[...], da_ref[...], db_ref[...],
                pexp_ref[...], b_ref[...])
    xl_out[...] = jnp.dot(h, wl_ref[...], preferred_element_type=_f32)
    xr_out[...] = jnp.dot(h, wr_ref[...], preferred_element_type=_f32)


_tc_dense2 = pl.pallas_call(
    _tc_dense2_body,
    out_shape=[jax.ShapeDtypeStruct((N, HID), _f32),
               jax.ShapeDtypeStruct((N, HID), _f32)],
)


def _tc_class_body(na_ref, nb_ref, da_ref, db_ref, pexp_ref, b_ref,
                   wc_ref, bc_ref, out_ref):
    h = _post_h(na_ref[...], nb_ref[...], da_ref[...], db_ref[...],
                pexp_ref[...], b_ref[...])
    out_ref[...] = jnp.dot(h, wc_ref[...], preferred_element_type=_f32) + bc_ref[...]


_tc_class = pl.pallas_call(
    _tc_class_body,
    out_shape=jax.ShapeDtypeStruct((N, OUT), _f32),
)


def _make_att_blockdiag(att):
    eye8 = jnp.eye(H, dtype=_f32)
    a = (att[:, :, None] * eye8[:, None, :]).reshape(HID, H)
    return jnp.pad(a, ((0, 0), (0, 16 - H)))


def _make_head_expand():
    eye8 = jnp.eye(H, dtype=_f32)
    p = (eye8[:, :, None] * jnp.ones((1, 1, C), _f32)).reshape(H, HID)
    return jnp.pad(p, ((0, 16 - H), (0, 0)))


def _layer(xl, xr, prep, We, a, pexp):
    src2, dst2, eap = prep
    xls, xrd = _sc_gather(xl, xr, src2, dst2)
    sc, ex = _tc_edge(xls, xrd, eap, We, a, pexp)
    num = _sc_num(sc, dst2).reshape(NC, NPADN, HID)
    den = _sc_den(ex, dst2).reshape(NC, NPADN, HID)[:, :, :16]
    return num, den


def kernel(x, edge_index, edge_attr, Wl1, Wr1, We1, att1, b1,
           Wl2, Wr2, We2, att2, b2, Wc, bc):
    srcp = jnp.pad(edge_index[0], (0, EPAD - E))
    dstp = jnp.pad(edge_index[1], (0, EPAD - E))
    src2 = srcp.reshape(EPAD // CHUNK, CHUNK)
    dst2 = dstp.reshape(EPAD // CHUNK, CHUNK)
    eap = jnp.pad(edge_attr, ((0, EPAD - E), (0, 0)))
    prep = (src2, dst2, eap)
    pexp = _make_head_expand()
    a1 = _make_att_blockdiag(att1)
    a2 = _make_att_blockdiag(att2)
    b1r = b1.reshape(1, HID)
    b2r = b2.reshape(1, HID)
    bcr = bc.reshape(1, OUT)

    xl1, xr1 = _tc_dense1(x, Wl1, Wr1)
    num1, den1 = _layer(xl1, xr1, prep, We1, a1, pexp)
    xl2, xr2 = _tc_dense2(num1[0, :N], num1[1, :N], den1[0, :N], den1[1, :N],
                          pexp, b1r, Wl2, Wr2)
    num2, den2 = _layer(xl2, xr2, prep, We2, a2, pexp)
    return _tc_class(num2[0, :N], num2[1, :N], den2[0, :N], den2[1, :N],
                     pexp, b2r, Wc, bcr)
